# baseline (speedup 1.0000x reference)
import numpy as np
import jax
import jax.numpy as jnp
from jax.experimental import pallas as pl

_MAX_LEN = 2048


def _pos_table(max_len, E):
    pos = np.arange(max_len, dtype=np.float64)[:, None]
    i = np.arange(E, dtype=np.float64)[None, :]
    angle = pos / np.power(10000.0, (i - np.mod(i, 2)) / E)
    angle[:, 0::2] = np.sin(angle[:, 0::2])
    angle[:, 1::2] = np.cos(angle[:, 1::2])
    return jnp.asarray(angle, dtype=jnp.float32)


def _tc_body(x_ref, pos_ref, o_ref):
    o_ref[0] = jnp.where(x_ref[0] == 0.0, 0.0, pos_ref[...])


def kernel(x):
    B, S, E = x.shape
    pos = _pos_table(_MAX_LEN, E)[:S]
    BLK = 256
    assert S % BLK == 0
    return pl.pallas_call(
        _tc_body,
        grid=(S // BLK, B),
        in_specs=[
            pl.BlockSpec((1, BLK, E), lambda s, b: (b, s, 0)),
            pl.BlockSpec((BLK, E), lambda s, b: (s, 0)),
        ],
        out_specs=pl.BlockSpec((1, BLK, E), lambda s, b: (b, s, 0)),
        out_shape=jax.ShapeDtypeStruct((B, S, E), jnp.float32),
    )(x, pos)
